# Initial kernel scaffold; baseline (speedup 1.0000x reference)
import jax
import jax.numpy as jnp
from jax.experimental import pallas as pl


def kernel(memory, last_update, unique_node_ids, unique_messages, timestamps, W_ih, W_hh, b_ih, b_hh):
    raise NotImplementedError("write your pallas kernel here")


# speedup vs baseline: 11.5284x; 11.5284x over previous
import jax
import jax.numpy as jnp
from jax import lax
from jax.experimental import pallas as pl
from jax.experimental.pallas import tpu as pltpu
from jax.experimental.pallas import tpu_sc as plsc

NC = 2
NS = 16
NW = NC * NS
CHUNK = 128
LANES = 16


def _sc_mesh():
    return plsc.VectorSubcoreMesh(core_axis_name="c", subcore_axis_name="s")


def _wid():
    return lax.axis_index("s") * NC + lax.axis_index("c")


def _sc_gather(memory, lu_blk, idx2):
    M, D = memory.shape
    k = idx2.shape[0] // NW
    bpw = k * CHUNK
    B = NW * bpw

    def body(mem_hbm, lublk_hbm, idx_hbm, h_out, lu_out,
             idx_v, idxblk_v, rows_v, lublk_v, lu_v, sem_r, sem_l):
        w = _wid()
        base = w * bpw
        pltpu.sync_copy(idx_hbm.at[pl.ds(w * k, k)], idx_v)
        row_copies = [
            pltpu.async_copy(mem_hbm.at[idx_v.at[j]],
                             rows_v.at[pl.ds(j * CHUNK, CHUNK)], sem_r)
            for j in range(k)
        ]
        for j in range(k):
            for g in range(CHUNK // LANES):
                v = idx_v[j, pl.ds(g * LANES, LANES)]
                idxblk_v[j, pl.ds(g * LANES, LANES)] = lax.shift_right_logical(
                    v, 7)
        iota = lax.iota(jnp.int32, LANES)
        for j in range(k):
            pltpu.async_copy(lublk_hbm.at[idxblk_v.at[j]], lublk_v,
                             sem_l).wait()
            for g in range(CHUNK // LANES):
                cols = idx_v[j, pl.ds(g * LANES, LANES)] & 127
                rowi = iota + (g * LANES)
                vals = plsc.load_gather(lublk_v, [rowi, cols])
                lu_v[pl.ds(j * CHUNK + g * LANES, LANES)] = vals
        for c in row_copies:
            c.wait()
        pltpu.sync_copy(rows_v, h_out.at[pl.ds(base, bpw)])
        pltpu.sync_copy(lu_v, lu_out.at[pl.ds(base, bpw)])

    f = pl.kernel(
        body,
        out_type=(jax.ShapeDtypeStruct((B, D), jnp.float32),
                  jax.ShapeDtypeStruct((B,), jnp.float32)),
        mesh=_sc_mesh(),
        scratch_types=[
            pltpu.VMEM((k, CHUNK), jnp.int32),
            pltpu.VMEM((k, CHUNK), jnp.int32),
            pltpu.VMEM((bpw, D), jnp.float32),
            pltpu.VMEM((CHUNK, 128), jnp.float32),
            pltpu.VMEM((bpw,), jnp.float32),
            pltpu.SemaphoreType.DMA,
            pltpu.SemaphoreType.DMA,
        ],
        compiler_params=pltpu.CompilerParams(needs_layout_passes=False),
        name="sc_gather_rows",
    )
    return f(memory, lu_blk, idx2)


def _sc_scatter(mem_ref, last_update, idx2, upd_rows, upd_ts):
    D = upd_rows.shape[1]
    M = last_update.shape[0]
    krows = idx2.shape[0]
    k = krows // NW
    bpw = k * CHUNK
    B = NW * bpw
    spt = M // NW
    GPR = CHUNK // LANES

    def body(mem_hbm, lu_hbm, idx_hbm, rows_hbm, ts_hbm, lu_out,
             idx_v, rows_v, idsf_v, ts_v, slice_v, sem_m):
        w = _wid()
        base = w * bpw
        lo = w * spt
        pltpu.sync_copy(idx_hbm.at[pl.ds(w * k, k)], idx_v)
        pltpu.sync_copy(rows_hbm.at[pl.ds(base, bpw)], rows_v)
        row_copies = [
            pltpu.async_copy(rows_v.at[pl.ds(j * CHUNK, CHUNK)],
                             mem_hbm.at[idx_v.at[j]], sem_m)
            for j in range(k)
        ]
        pltpu.sync_copy(lu_hbm.at[pl.ds(lo, spt)], slice_v)
        pltpu.sync_copy(idx_hbm, idsf_v)
        pltpu.sync_copy(ts_hbm, ts_v)

        @pl.loop(0, krows)
        def _(r):
            for g in range(GPR):
                ids16 = idsf_v[r, pl.ds(g * LANES, LANES)]
                m = (ids16 >= lo) & (ids16 < lo + spt)
                loc = jnp.minimum(jnp.maximum(ids16 - lo, 0), spt - 1)
                vals = ts_v[r, pl.ds(g * LANES, LANES)]
                plsc.store_scatter(slice_v, [loc], vals, mask=m)

        pltpu.sync_copy(slice_v, lu_out.at[pl.ds(lo, spt)])
        for c in row_copies:
            c.wait()

    f = pl.kernel(
        body,
        out_type=jax.ShapeDtypeStruct((M,), jnp.float32),
        mesh=_sc_mesh(),
        scratch_types=[
            pltpu.VMEM((k, CHUNK), jnp.int32),
            pltpu.VMEM((bpw, D), jnp.float32),
            pltpu.VMEM((krows, CHUNK), jnp.int32),
            pltpu.VMEM((krows, CHUNK), jnp.float32),
            pltpu.VMEM((spt,), jnp.float32),
            pltpu.SemaphoreType.DMA,
        ],
        compiler_params=pltpu.CompilerParams(needs_layout_passes=False),
        name="sc_scatter_rows",
    )
    return f(mem_ref, last_update, idx2, upd_rows, upd_ts)


def _tc_gru(messages, h, lu_g, ts, W_ih, W_hh, b_ih, b_hh):
    B, _ = messages.shape
    D = h.shape[1]
    R = 2048
    G = B // R
    luc = lu_g.reshape(B, 1)
    tsc = ts.reshape(B, 1)
    bih2 = b_ih.reshape(1, 3 * D)
    bhh2 = b_hh.reshape(1, 3 * D)

    def body(x_ref, h_ref, lu_ref, ts_ref, wih_ref, whh_ref, bih_ref, bhh_ref,
             out_ref, ts_out_ref):
        x = x_ref[...]
        hh = h_ref[...]
        dn = (((1,), (1,)), ((), ()))
        gi = lax.dot_general(x, wih_ref[...], dimension_numbers=dn,
                             preferred_element_type=jnp.float32) + bih_ref[...]
        gh = lax.dot_general(hh, whh_ref[...], dimension_numbers=dn,
                             preferred_element_type=jnp.float32) + bhh_ref[...]
        r = jax.nn.sigmoid(gi[:, :D] + gh[:, :D])
        z = jax.nn.sigmoid(gi[:, D:2 * D] + gh[:, D:2 * D])
        n = jnp.tanh(gi[:, 2 * D:] + r * gh[:, 2 * D:])
        h_new = (1.0 - z) * n + z * hh
        lu = lu_ref[...]
        tss = ts_ref[...]
        valid = lu <= tss
        out_ref[...] = jnp.where(valid, h_new, hh)
        ts_out_ref[...] = jnp.maximum(tss, lu)

    return pl.pallas_call(
        body,
        grid=(G,),
        in_specs=[
            pl.BlockSpec((R, messages.shape[1]), lambda g: (g, 0)),
            pl.BlockSpec((R, D), lambda g: (g, 0)),
            pl.BlockSpec((R, 1), lambda g: (g, 0)),
            pl.BlockSpec((R, 1), lambda g: (g, 0)),
            pl.BlockSpec(W_ih.shape, lambda g: (0, 0)),
            pl.BlockSpec(W_hh.shape, lambda g: (0, 0)),
            pl.BlockSpec((1, 3 * D), lambda g: (0, 0)),
            pl.BlockSpec((1, 3 * D), lambda g: (0, 0)),
        ],
        out_specs=[
            pl.BlockSpec((R, D), lambda g: (g, 0)),
            pl.BlockSpec((R, 1), lambda g: (g, 0)),
        ],
        out_shape=[
            jax.ShapeDtypeStruct((B, D), jnp.float32),
            jax.ShapeDtypeStruct((B, 1), jnp.float32),
        ],
        name="tc_gru_cell",
    )(messages, h, luc, tsc, W_ih, W_hh, bih2, bhh2)


def kernel(memory, last_update, unique_node_ids, unique_messages, timestamps,
           W_ih, W_hh, b_ih, b_hh):
    M, D = memory.shape
    B = unique_messages.shape[0]
    ids = unique_node_ids.astype(jnp.int32)
    idx2 = ids.reshape(NW * (B // NW // CHUNK), CHUNK)
    lu_blk = last_update.reshape(M // 128, 128)

    h, lu_g = _sc_gather(memory, lu_blk, idx2)
    upd_rows, upd_ts3 = _tc_gru(unique_messages, h, lu_g, timestamps,
                                W_ih, W_hh, b_ih, b_hh)

    mem_ref = jax.new_ref(memory)
    out_lu = _sc_scatter(mem_ref, last_update, idx2, upd_rows,
                         upd_ts3.reshape(idx2.shape))
    return mem_ref[...], out_lu


# speedup vs baseline: 11.7017x; 1.0150x over previous
import jax
import jax.numpy as jnp
from jax import lax
from jax.experimental import pallas as pl
from jax.experimental.pallas import tpu as pltpu
from jax.experimental.pallas import tpu_sc as plsc

NC = 2
NS = 16
NW = NC * NS
CHUNK = 128
LANES = 16


def _sc_mesh():
    return plsc.VectorSubcoreMesh(core_axis_name="c", subcore_axis_name="s")


def _sc_gather_ts(memory, last_update, ids, ts):
    M, D = memory.shape
    B = ids.shape[0]
    k = B // NW // CHUNK
    bpw = k * CHUNK
    spt = M // NW
    GPR = CHUNK // LANES
    BR = B // CHUNK

    def body(mem_hbm, lu_hbm, ids_hbm, ts_hbm, h_out, parts_out, lu_out,
             idx_v, rows_v, ids_sc, ts_sc, slice_v, part_v, iota_v, shared,
             sem_r):
        c = lax.axis_index("c")
        s = lax.axis_index("s")
        w = s * NC + c
        base = w * bpw
        lo = w * spt
        for j in range(k):
            pltpu.sync_copy(ids_hbm.at[pl.ds(base + j * CHUNK, CHUNK)],
                            idx_v.at[j])
        row_copies = [
            pltpu.async_copy(mem_hbm.at[idx_v.at[j]],
                             rows_v.at[pl.ds(j * CHUNK, CHUNK)], sem_r)
            for j in range(k)
        ]
        pltpu.sync_copy(lu_hbm.at[pl.ds(lo, spt)], slice_v)
        pltpu.sync_copy(ids_hbm, ids_sc)
        pltpu.sync_copy(ts_hbm, ts_sc)
        iota = lax.iota(jnp.int32, LANES)
        for g in range(BR // LANES):
            iota_v[pl.ds(g * LANES, LANES)] = iota + (g * LANES)

        @pl.loop(0, BR)
        def _(r):
            for u in range(GPR):
                o = r * CHUNK + u * LANES
                ids16 = ids_sc[pl.ds(o, LANES)]
                m = (ids16 >= lo) & (ids16 < lo + spt)
                loc = jnp.minimum(jnp.maximum(ids16 - lo, 0), spt - 1)
                cur = plsc.load_gather(slice_v, [loc], mask=m)
                part_v[r, pl.ds(u * LANES, LANES)] = jnp.where(m, cur, 0.0)
                newv = jnp.maximum(ts_sc[pl.ds(o, LANES)], cur)
                plsc.store_scatter(slice_v, [loc], newv, mask=m)

        pltpu.sync_copy(slice_v, lu_out.at[pl.ds(lo, spt)])
        @pl.when(s == 0)
        def _():
            pltpu.sync_copy(part_v, shared)
        plsc.subcore_barrier()

        @pl.when(s != 0)
        def _():
            pltpu.sync_copy(part_v, shared.at[iota_v], add=True)
        plsc.subcore_barrier()

        @pl.when(s == 0)
        def _():
            pltpu.sync_copy(shared, parts_out.at[c])
        for cp in row_copies:
            cp.wait()
        pltpu.sync_copy(rows_v, h_out.at[pl.ds(base, bpw)])

    f = pl.kernel(
        body,
        out_type=(jax.ShapeDtypeStruct((B, D), jnp.float32),
                  jax.ShapeDtypeStruct((NC, BR, CHUNK), jnp.float32),
                  jax.ShapeDtypeStruct((M,), jnp.float32)),
        mesh=_sc_mesh(),
        scratch_types=[
            pltpu.VMEM((k, CHUNK), jnp.int32),
            pltpu.VMEM((bpw, D), jnp.float32),
            pltpu.VMEM((B,), jnp.int32),
            pltpu.VMEM((B,), jnp.float32),
            pltpu.VMEM((spt,), jnp.float32),
            pltpu.VMEM((BR, CHUNK), jnp.float32),
            pltpu.VMEM((BR,), jnp.int32),
            pltpu.VMEM_SHARED((BR, CHUNK), jnp.float32),
            pltpu.SemaphoreType.DMA,
        ],
        compiler_params=pltpu.CompilerParams(needs_layout_passes=False),
        name="sc_gather_ts",
    )
    return f(memory, last_update, ids, ts)


def _sc_scatter(mem_ref, ids, upd_rows):
    B, D = upd_rows.shape
    k = B // NW // CHUNK
    bpw = k * CHUNK

    def body(mem_hbm, ids_hbm, rows_hbm, idx_v, rows_v, sem_m):
        c = lax.axis_index("c")
        s = lax.axis_index("s")
        w = s * NC + c
        base = w * bpw
        for j in range(k):
            pltpu.sync_copy(ids_hbm.at[pl.ds(base + j * CHUNK, CHUNK)],
                            idx_v.at[j])
        pltpu.sync_copy(rows_hbm.at[pl.ds(base, bpw)], rows_v)
        row_copies = [
            pltpu.async_copy(rows_v.at[pl.ds(j * CHUNK, CHUNK)],
                             mem_hbm.at[idx_v.at[j]], sem_m)
            for j in range(k)
        ]
        for cp in row_copies:
            cp.wait()

    f = pl.kernel(
        body,
        out_type=(),
        mesh=_sc_mesh(),
        scratch_types=[
            pltpu.VMEM((k, CHUNK), jnp.int32),
            pltpu.VMEM((bpw, D), jnp.float32),
            pltpu.SemaphoreType.DMA,
        ],
        compiler_params=pltpu.CompilerParams(needs_layout_passes=False),
        name="sc_scatter_rows",
    )
    f(mem_ref, ids, upd_rows)


def _tc_gru(messages, h, lu_col, ts_col, W_ih, W_hh, b_ih, b_hh):
    B, _ = messages.shape
    D = h.shape[1]
    R = 2048
    G = B // R
    bih2 = b_ih.reshape(1, 3 * D)
    bhh2 = b_hh.reshape(1, 3 * D)

    def body(x_ref, h_ref, lu_ref, ts_ref, wih_ref, whh_ref, bih_ref, bhh_ref,
             out_ref):
        x = x_ref[...]
        hh = h_ref[...]
        dn = (((1,), (1,)), ((), ()))
        gi = lax.dot_general(x, wih_ref[...], dimension_numbers=dn,
                             preferred_element_type=jnp.float32) + bih_ref[...]
        gh = lax.dot_general(hh, whh_ref[...], dimension_numbers=dn,
                             preferred_element_type=jnp.float32) + bhh_ref[...]
        r = jax.nn.sigmoid(gi[:, :D] + gh[:, :D])
        z = jax.nn.sigmoid(gi[:, D:2 * D] + gh[:, D:2 * D])
        n = jnp.tanh(gi[:, 2 * D:] + r * gh[:, 2 * D:])
        h_new = (1.0 - z) * n + z * hh
        valid = lu_ref[...] <= ts_ref[...]
        out_ref[...] = jnp.where(valid, h_new, hh)

    return pl.pallas_call(
        body,
        grid=(G,),
        in_specs=[
            pl.BlockSpec((R, messages.shape[1]), lambda g: (g, 0)),
            pl.BlockSpec((R, D), lambda g: (g, 0)),
            pl.BlockSpec((R, 1), lambda g: (g, 0)),
            pl.BlockSpec((R, 1), lambda g: (g, 0)),
            pl.BlockSpec(W_ih.shape, lambda g: (0, 0)),
            pl.BlockSpec(W_hh.shape, lambda g: (0, 0)),
            pl.BlockSpec((1, 3 * D), lambda g: (0, 0)),
            pl.BlockSpec((1, 3 * D), lambda g: (0, 0)),
        ],
        out_specs=pl.BlockSpec((R, D), lambda g: (g, 0)),
        out_shape=jax.ShapeDtypeStruct((B, D), jnp.float32),
        name="tc_gru_cell",
    )(messages, h, lu_col, ts_col, W_ih, W_hh, bih2, bhh2)


def kernel(memory, last_update, unique_node_ids, unique_messages, timestamps,
           W_ih, W_hh, b_ih, b_hh):
    B = unique_messages.shape[0]
    ids = unique_node_ids.astype(jnp.int32)

    h, lu_parts, out_lu = _sc_gather_ts(memory, last_update, ids, timestamps)
    lu_col = (lu_parts[0] + lu_parts[1]).reshape(B, 1)
    ts_col = timestamps.reshape(B, 1)
    upd_rows = _tc_gru(unique_messages, h, lu_col, ts_col,
                       W_ih, W_hh, b_ih, b_hh)

    mem_ref = jax.new_ref(memory)
    _sc_scatter(mem_ref, ids, upd_rows)
    return mem_ref[...], out_lu


# speedup vs baseline: 11.7075x; 1.0005x over previous
import jax
import jax.numpy as jnp
from jax import lax
from jax.experimental import pallas as pl
from jax.experimental.pallas import tpu as pltpu
from jax.experimental.pallas import tpu_sc as plsc

NC = 2
NS = 16
NW = NC * NS
CHUNK = 128
LANES = 16


def _sc_mesh():
    return plsc.VectorSubcoreMesh(core_axis_name="c", subcore_axis_name="s")


def _sc_gather_ts(memory, last_update, ids, ts):
    M, D = memory.shape
    B = ids.shape[0]
    k = B // NW // CHUNK
    bpw = k * CHUNK
    spt = M // NW
    GPR = CHUNK // LANES
    BR = B // CHUNK

    def body(mem_hbm, lu_hbm, ids_hbm, ts_hbm, h_out, parts_out, lu_out,
             idx_v, rows_v, ids_sc, ts_sc, slice_v, part_v, iota_v, shared,
             sem_r):
        c = lax.axis_index("c")
        s = lax.axis_index("s")
        w = s * NC + c
        base = w * bpw
        lo = w * spt
        for j in range(k):
            pltpu.sync_copy(ids_hbm.at[pl.ds(base + j * CHUNK, CHUNK)],
                            idx_v.at[j])
        row_copies = [
            pltpu.async_copy(mem_hbm.at[idx_v.at[j]],
                             rows_v.at[pl.ds(j * CHUNK, CHUNK)], sem_r)
            for j in range(k)
        ]
        pltpu.sync_copy(lu_hbm.at[pl.ds(lo, spt)], slice_v)
        pltpu.sync_copy(ids_hbm, ids_sc)
        pltpu.sync_copy(ts_hbm, ts_sc)
        iota = lax.iota(jnp.int32, LANES)
        for g in range(BR // LANES):
            iota_v[pl.ds(g * LANES, LANES)] = iota + (g * LANES)

        @pl.loop(0, BR)
        def _(r):
            for u in range(GPR):
                o = r * CHUNK + u * LANES
                ids16 = ids_sc[pl.ds(o, LANES)]
                m = (ids16 >= lo) & (ids16 < lo + spt)
                loc = jnp.minimum(jnp.maximum(ids16 - lo, 0), spt - 1)
                cur = plsc.load_gather(slice_v, [loc], mask=m)
                part_v[r, pl.ds(u * LANES, LANES)] = jnp.where(m, cur, 0.0)
                newv = jnp.maximum(ts_sc[pl.ds(o, LANES)], cur)
                plsc.store_scatter(slice_v, [loc], newv, mask=m)

        pltpu.sync_copy(slice_v, lu_out.at[pl.ds(lo, spt)])
        @pl.when(s == 0)
        def _():
            pltpu.sync_copy(part_v, shared)
        plsc.subcore_barrier()

        @pl.when(s != 0)
        def _():
            pltpu.sync_copy(part_v, shared.at[iota_v], add=True)
        plsc.subcore_barrier()

        @pl.when(s == 0)
        def _():
            pltpu.sync_copy(shared, parts_out.at[c])
        for cp in row_copies:
            cp.wait()
        pltpu.sync_copy(rows_v, h_out.at[pl.ds(base, bpw)])

    f = pl.kernel(
        body,
        out_type=(jax.ShapeDtypeStruct((B, D), jnp.float32),
                  jax.ShapeDtypeStruct((NC, BR, CHUNK), jnp.float32),
                  jax.ShapeDtypeStruct((M,), jnp.float32)),
        mesh=_sc_mesh(),
        scratch_types=[
            pltpu.VMEM((k, CHUNK), jnp.int32),
            pltpu.VMEM((bpw, D), jnp.float32),
            pltpu.VMEM((B,), jnp.int32),
            pltpu.VMEM((B,), jnp.float32),
            pltpu.VMEM((spt,), jnp.float32),
            pltpu.VMEM((BR, CHUNK), jnp.float32),
            pltpu.VMEM((BR,), jnp.int32),
            pltpu.VMEM_SHARED((BR, CHUNK), jnp.float32),
            pltpu.SemaphoreType.DMA,
        ],
        compiler_params=pltpu.CompilerParams(needs_layout_passes=False),
        cost_estimate=pl.CostEstimate(
            flops=0, bytes_accessed=400_000_000, transcendentals=0),
        name="sc_gather_ts",
    )
    return f(memory, last_update, ids, ts)


def _sc_scatter(mem_ref, ids, upd_rows):
    B, D = upd_rows.shape
    k = B // NW // CHUNK
    bpw = k * CHUNK

    def body(mem_hbm, ids_hbm, rows_hbm, idx_v, rows_v, sem_m):
        c = lax.axis_index("c")
        s = lax.axis_index("s")
        w = s * NC + c
        base = w * bpw
        for j in range(k):
            pltpu.sync_copy(ids_hbm.at[pl.ds(base + j * CHUNK, CHUNK)],
                            idx_v.at[j])
        pltpu.sync_copy(rows_hbm.at[pl.ds(base, bpw)], rows_v)
        row_copies = [
            pltpu.async_copy(rows_v.at[pl.ds(j * CHUNK, CHUNK)],
                             mem_hbm.at[idx_v.at[j]], sem_m)
            for j in range(k)
        ]
        for cp in row_copies:
            cp.wait()

    f = pl.kernel(
        body,
        out_type=(),
        mesh=_sc_mesh(),
        scratch_types=[
            pltpu.VMEM((k, CHUNK), jnp.int32),
            pltpu.VMEM((bpw, D), jnp.float32),
            pltpu.SemaphoreType.DMA,
        ],
        compiler_params=pltpu.CompilerParams(needs_layout_passes=False),
        name="sc_scatter_rows",
    )
    f(mem_ref, ids, upd_rows)


def _tc_gru(messages, h, lu_col, ts_col, W_ih, W_hh, b_ih, b_hh):
    B, _ = messages.shape
    D = h.shape[1]
    R = 2048
    G = B // R
    bih2 = b_ih.reshape(1, 3 * D)
    bhh2 = b_hh.reshape(1, 3 * D)

    def body(x_ref, h_ref, lu_ref, ts_ref, wih_ref, whh_ref, bih_ref, bhh_ref,
             out_ref):
        x = x_ref[...]
        hh = h_ref[...]
        dn = (((1,), (1,)), ((), ()))
        gi = lax.dot_general(x, wih_ref[...], dimension_numbers=dn,
                             preferred_element_type=jnp.float32) + bih_ref[...]
        gh = lax.dot_general(hh, whh_ref[...], dimension_numbers=dn,
                             preferred_element_type=jnp.float32) + bhh_ref[...]
        r = jax.nn.sigmoid(gi[:, :D] + gh[:, :D])
        z = jax.nn.sigmoid(gi[:, D:2 * D] + gh[:, D:2 * D])
        n = jnp.tanh(gi[:, 2 * D:] + r * gh[:, 2 * D:])
        h_new = (1.0 - z) * n + z * hh
        valid = lu_ref[...] <= ts_ref[...]
        out_ref[...] = jnp.where(valid, h_new, hh)

    return pl.pallas_call(
        body,
        grid=(G,),
        in_specs=[
            pl.BlockSpec((R, messages.shape[1]), lambda g: (g, 0)),
            pl.BlockSpec((R, D), lambda g: (g, 0)),
            pl.BlockSpec((R, 1), lambda g: (g, 0)),
            pl.BlockSpec((R, 1), lambda g: (g, 0)),
            pl.BlockSpec(W_ih.shape, lambda g: (0, 0)),
            pl.BlockSpec(W_hh.shape, lambda g: (0, 0)),
            pl.BlockSpec((1, 3 * D), lambda g: (0, 0)),
            pl.BlockSpec((1, 3 * D), lambda g: (0, 0)),
        ],
        out_specs=pl.BlockSpec((R, D), lambda g: (g, 0)),
        out_shape=jax.ShapeDtypeStruct((B, D), jnp.float32),
        name="tc_gru_cell",
    )(messages, h, lu_col, ts_col, W_ih, W_hh, bih2, bhh2)


def kernel(memory, last_update, unique_node_ids, unique_messages, timestamps,
           W_ih, W_hh, b_ih, b_hh):
    B = unique_messages.shape[0]
    ids = unique_node_ids.astype(jnp.int32)

    h, lu_parts, out_lu = _sc_gather_ts(memory, last_update, ids, timestamps)
    lu_col = (lu_parts[0] + lu_parts[1]).reshape(B, 1)
    ts_col = timestamps.reshape(B, 1)
    upd_rows = _tc_gru(unique_messages, h, lu_col, ts_col,
                       W_ih, W_hh, b_ih, b_hh)

    mem_ref = jax.new_ref(memory)
    _sc_scatter(mem_ref, ids, upd_rows)
    return mem_ref[...], out_lu


# speedup vs baseline: 12.0624x; 1.0303x over previous
import jax
import jax.numpy as jnp
from jax import lax
from jax.experimental import pallas as pl
from jax.experimental.pallas import tpu as pltpu
from jax.experimental.pallas import tpu_sc as plsc

NC = 2
NS = 16
NW = NC * NS
CHUNK = 128
LANES = 16


def _sc_mesh():
    return plsc.VectorSubcoreMesh(core_axis_name="c", subcore_axis_name="s")


def _sc_gather_ts(memory, lu_blk, ids, ts):
    M2, _ = lu_blk.shape
    M, D = memory.shape
    B = ids.shape[0]
    k = B // NW // CHUNK
    bpw = k * CHUNK
    srows = M2 // NW
    spt = srows * CHUNK
    GPR = CHUNK // LANES

    def body(mem_hbm, lublk_hbm, ids_hbm, ts_hbm, h_out, lu_out, lublk_out,
             idx_v, idxblk_v, rows_v, lublk_v, lu_v, ids_sc, ts_sc, slice_v,
             sem_r, sem_l):
        c = lax.axis_index("c")
        s = lax.axis_index("s")
        w = s * NC + c
        base = w * bpw
        lo = w * spt
        for j in range(k):
            pltpu.sync_copy(ids_hbm.at[pl.ds(base + j * CHUNK, CHUNK)],
                            idx_v.at[j])
        row_copies = [
            pltpu.async_copy(mem_hbm.at[idx_v.at[j]],
                             rows_v.at[pl.ds(j * CHUNK, CHUNK)], sem_r)
            for j in range(k)
        ]
        iota = lax.iota(jnp.int32, LANES)
        for j in range(k):
            for g in range(GPR):
                v = idx_v[j, pl.ds(g * LANES, LANES)]
                idxblk_v[j, pl.ds(g * LANES, LANES)] = (
                    lax.shift_right_logical(v, 7))
        for j in range(k):
            pltpu.async_copy(lublk_hbm.at[idxblk_v.at[j]], lublk_v,
                             sem_l).wait()
            for g in range(GPR):
                cols = idx_v[j, pl.ds(g * LANES, LANES)] & 127
                rowi = iota + (g * LANES)
                vals = plsc.load_gather(lublk_v, [rowi, cols])
                lu_v[pl.ds(j * CHUNK + g * LANES, LANES)] = vals
        pltpu.sync_copy(lublk_hbm.at[pl.ds(w * srows, srows)], slice_v)
        pltpu.sync_copy(ids_hbm, ids_sc)
        pltpu.sync_copy(ts_hbm, ts_sc)

        @plsc.parallel_loop(0, B // LANES, unroll=4)
        def _(g):
            ids16 = ids_sc[pl.ds(g * LANES, LANES)]
            m = (ids16 >= lo) & (ids16 < lo + spt)
            loc = jnp.minimum(jnp.maximum(ids16 - lo, 0), spt - 1)
            locr = lax.shift_right_logical(loc, 7)
            locc = loc & 127
            cur = plsc.load_gather(slice_v, [locr, locc], mask=m)
            newv = jnp.maximum(ts_sc[pl.ds(g * LANES, LANES)], cur)
            plsc.store_scatter(slice_v, [locr, locc], newv, mask=m)

        pltpu.sync_copy(slice_v, lublk_out.at[pl.ds(w * srows, srows)])
        for cp in row_copies:
            cp.wait()
        pltpu.sync_copy(rows_v, h_out.at[pl.ds(base, bpw)])
        pltpu.sync_copy(lu_v, lu_out.at[pl.ds(base, bpw)])

    f = pl.kernel(
        body,
        out_type=(jax.ShapeDtypeStruct((B, D), jnp.float32),
                  jax.ShapeDtypeStruct((B,), jnp.float32),
                  jax.ShapeDtypeStruct((M2, CHUNK), jnp.float32)),
        mesh=_sc_mesh(),
        scratch_types=[
            pltpu.VMEM((k, CHUNK), jnp.int32),
            pltpu.VMEM((k, CHUNK), jnp.int32),
            pltpu.VMEM((bpw, D), jnp.float32),
            pltpu.VMEM((CHUNK, CHUNK), jnp.float32),
            pltpu.VMEM((bpw,), jnp.float32),
            pltpu.VMEM((B,), jnp.int32),
            pltpu.VMEM((B,), jnp.float32),
            pltpu.VMEM((srows, CHUNK), jnp.float32),
            pltpu.SemaphoreType.DMA,
            pltpu.SemaphoreType.DMA,
        ],
        compiler_params=pltpu.CompilerParams(needs_layout_passes=False),
        name="sc_gather_ts",
    )
    return f(memory, lu_blk, ids, ts)


def _sc_scatter(mem_ref, ids, upd_rows):
    B, D = upd_rows.shape
    k = B // NW // CHUNK
    bpw = k * CHUNK

    def body(mem_hbm, ids_hbm, rows_hbm, idx_v, rows_v, sem_m):
        c = lax.axis_index("c")
        s = lax.axis_index("s")
        w = s * NC + c
        base = w * bpw
        for j in range(k):
            pltpu.sync_copy(ids_hbm.at[pl.ds(base + j * CHUNK, CHUNK)],
                            idx_v.at[j])
        pltpu.sync_copy(rows_hbm.at[pl.ds(base, bpw)], rows_v)
        row_copies = [
            pltpu.async_copy(rows_v.at[pl.ds(j * CHUNK, CHUNK)],
                             mem_hbm.at[idx_v.at[j]], sem_m)
            for j in range(k)
        ]
        for cp in row_copies:
            cp.wait()

    f = pl.kernel(
        body,
        out_type=(),
        mesh=_sc_mesh(),
        scratch_types=[
            pltpu.VMEM((k, CHUNK), jnp.int32),
            pltpu.VMEM((bpw, D), jnp.float32),
            pltpu.SemaphoreType.DMA,
        ],
        compiler_params=pltpu.CompilerParams(needs_layout_passes=False),
        name="sc_scatter_rows",
    )
    f(mem_ref, ids, upd_rows)


def _tc_copy_gru(memory, messages, h, lu_col, ts_col, W_ih, W_hh, b_ih, b_hh):
    M, D = memory.shape
    B, DM = messages.shape
    R = 2048
    G = B // R
    CB = 8192
    NB = M // CB
    bih2 = b_ih.reshape(1, 3 * D)
    bhh2 = b_hh.reshape(1, 3 * D)

    def body(mem_ref, x_ref, h_ref, lu_ref, ts_ref, wih_ref, whh_ref,
             bih_ref, bhh_ref, out_mem_ref, upd_ref):
        i = pl.program_id(0)

        @pl.when(i < G)
        def _():
            x = x_ref[...]
            hh = h_ref[...]
            dn = (((1,), (1,)), ((), ()))
            gi = lax.dot_general(
                x, wih_ref[...], dimension_numbers=dn,
                preferred_element_type=jnp.float32) + bih_ref[...]
            gh = lax.dot_general(
                hh, whh_ref[...], dimension_numbers=dn,
                preferred_element_type=jnp.float32) + bhh_ref[...]
            r = jax.nn.sigmoid(gi[:, :D] + gh[:, :D])
            z = jax.nn.sigmoid(gi[:, D:2 * D] + gh[:, D:2 * D])
            n = jnp.tanh(gi[:, 2 * D:] + r * gh[:, 2 * D:])
            h_new = (1.0 - z) * n + z * hh
            valid = lu_ref[...] <= ts_ref[...]
            upd_ref[...] = jnp.where(valid, h_new, hh)

        @pl.when(i >= G)
        def _():
            out_mem_ref[...] = mem_ref[...]

    return pl.pallas_call(
        body,
        grid=(G + NB,),
        in_specs=[
            pl.BlockSpec((CB, D), lambda i: (jnp.maximum(i - G, 0), 0)),
            pl.BlockSpec((R, DM), lambda i: (jnp.minimum(i, G - 1), 0)),
            pl.BlockSpec((R, D), lambda i: (jnp.minimum(i, G - 1), 0)),
            pl.BlockSpec((R, 1), lambda i: (jnp.minimum(i, G - 1), 0)),
            pl.BlockSpec((R, 1), lambda i: (jnp.minimum(i, G - 1), 0)),
            pl.BlockSpec(W_ih.shape, lambda i: (0, 0)),
            pl.BlockSpec(W_hh.shape, lambda i: (0, 0)),
            pl.BlockSpec((1, 3 * D), lambda i: (0, 0)),
            pl.BlockSpec((1, 3 * D), lambda i: (0, 0)),
        ],
        out_specs=[
            pl.BlockSpec((CB, D), lambda i: (jnp.maximum(i - G, 0), 0)),
            pl.BlockSpec((R, D), lambda i: (jnp.minimum(i, G - 1), 0)),
        ],
        out_shape=[
            jax.ShapeDtypeStruct((M, D), jnp.float32),
            jax.ShapeDtypeStruct((B, D), jnp.float32),
        ],
        name="tc_copy_gru",
    )(memory, messages, h, lu_col, ts_col, W_ih, W_hh, bih2, bhh2)


def kernel(memory, last_update, unique_node_ids, unique_messages, timestamps,
           W_ih, W_hh, b_ih, b_hh):
    M = memory.shape[0]
    B = unique_messages.shape[0]
    ids = unique_node_ids.astype(jnp.int32)
    lu_blk = last_update.reshape(M // 128, 128)

    h, lu_g, lu_out_blk = _sc_gather_ts(memory, lu_blk, ids, timestamps)
    lu_col = lu_g.reshape(B, 1)
    ts_col = timestamps.reshape(B, 1)
    out_mem0, upd_rows = _tc_copy_gru(memory, unique_messages, h, lu_col,
                                      ts_col, W_ih, W_hh, b_ih, b_hh)

    mem_ref = jax.new_ref(out_mem0)
    _sc_scatter(mem_ref, ids, upd_rows)
    return mem_ref[...], lu_out_blk.reshape(M)


# speedup vs baseline: 12.5095x; 1.0371x over previous
import jax
import jax.numpy as jnp
from jax import lax
from jax.experimental import pallas as pl
from jax.experimental.pallas import tpu as pltpu
from jax.experimental.pallas import tpu_sc as plsc

NC = 2
NS = 16
NW = NC * NS
CHUNK = 128
LANES = 16


def _sc_mesh():
    return plsc.VectorSubcoreMesh(core_axis_name="c", subcore_axis_name="s")


def _sc_gather_ts(memory, lu_blk, ids, ts):
    M2, _ = lu_blk.shape
    M, D = memory.shape
    B = ids.shape[0]
    k = B // NW // CHUNK
    bpw = k * CHUNK
    srows = M2 // NW
    spt = srows * CHUNK
    GPR = CHUNK // LANES

    def body(mem_hbm, lublk_hbm, ids_hbm, ts_hbm, h_out, lu_out, lublk_out,
             idx_v, idxblk_v, rows_v, lublk_v, lu_v, ids_sc, ts_sc, slice_v,
             sem_r, sem_l, sem_s):
        c = lax.axis_index("c")
        s = lax.axis_index("s")
        w = s * NC + c
        base = w * bpw
        lo = w * spt
        for j in range(k):
            pltpu.sync_copy(ids_hbm.at[pl.ds(base + j * CHUNK, CHUNK)],
                            idx_v.at[j])
        row_copies = [
            pltpu.async_copy(mem_hbm.at[idx_v.at[j]],
                             rows_v.at[pl.ds(j * CHUNK, CHUNK)], sem_r)
            for j in range(k)
        ]
        scan_copies = [
            pltpu.async_copy(lublk_hbm.at[pl.ds(w * srows, srows)], slice_v,
                             sem_s),
            pltpu.async_copy(ids_hbm, ids_sc, sem_s),
            pltpu.async_copy(ts_hbm, ts_sc, sem_s),
        ]
        iota = lax.iota(jnp.int32, LANES)
        for j in range(k):
            for g in range(GPR):
                v = idx_v[j, pl.ds(g * LANES, LANES)]
                idxblk_v[j, pl.ds(g * LANES, LANES)] = (
                    lax.shift_right_logical(v, 7))
        for j in range(k):
            pltpu.async_copy(lublk_hbm.at[idxblk_v.at[j]], lublk_v,
                             sem_l).wait()
            for g in range(GPR):
                cols = idx_v[j, pl.ds(g * LANES, LANES)] & 127
                rowi = iota + (g * LANES)
                vals = plsc.load_gather(lublk_v, [rowi, cols])
                lu_v[pl.ds(j * CHUNK + g * LANES, LANES)] = vals
        for cp in scan_copies:
            cp.wait()

        @plsc.parallel_loop(0, B // LANES, unroll=4)
        def _(g):
            ids16 = ids_sc[pl.ds(g * LANES, LANES)]
            m = (ids16 >= lo) & (ids16 < lo + spt)
            loc = jnp.minimum(jnp.maximum(ids16 - lo, 0), spt - 1)
            locr = lax.shift_right_logical(loc, 7)
            locc = loc & 127
            cur = plsc.load_gather(slice_v, [locr, locc], mask=m)
            newv = jnp.maximum(ts_sc[pl.ds(g * LANES, LANES)], cur)
            plsc.store_scatter(slice_v, [locr, locc], newv, mask=m)

        pltpu.sync_copy(slice_v, lublk_out.at[pl.ds(w * srows, srows)])
        for cp in row_copies:
            cp.wait()
        pltpu.sync_copy(rows_v, h_out.at[pl.ds(base, bpw)])
        pltpu.sync_copy(lu_v, lu_out.at[pl.ds(base, bpw)])

    f = pl.kernel(
        body,
        out_type=(jax.ShapeDtypeStruct((B, D), jnp.float32),
                  jax.ShapeDtypeStruct((B,), jnp.float32),
                  jax.ShapeDtypeStruct((M2, CHUNK), jnp.float32)),
        mesh=_sc_mesh(),
        scratch_types=[
            pltpu.VMEM((k, CHUNK), jnp.int32),
            pltpu.VMEM((k, CHUNK), jnp.int32),
            pltpu.VMEM((bpw, D), jnp.float32),
            pltpu.VMEM((CHUNK, CHUNK), jnp.float32),
            pltpu.VMEM((bpw,), jnp.float32),
            pltpu.VMEM((B,), jnp.int32),
            pltpu.VMEM((B,), jnp.float32),
            pltpu.VMEM((srows, CHUNK), jnp.float32),
            pltpu.SemaphoreType.DMA,
            pltpu.SemaphoreType.DMA,
            pltpu.SemaphoreType.DMA,
        ],
        compiler_params=pltpu.CompilerParams(needs_layout_passes=False),
        name="sc_gather_ts",
    )
    return f(memory, lu_blk, ids, ts)


def _sc_scatter(mem_ref, ids, upd_rows):
    B, D = upd_rows.shape
    k = B // NW // CHUNK
    bpw = k * CHUNK

    def body(mem_hbm, ids_hbm, rows_hbm, idx_v, rows_v, sem_m):
        c = lax.axis_index("c")
        s = lax.axis_index("s")
        w = s * NC + c
        base = w * bpw
        for j in range(k):
            pltpu.sync_copy(ids_hbm.at[pl.ds(base + j * CHUNK, CHUNK)],
                            idx_v.at[j])
        pltpu.sync_copy(rows_hbm.at[pl.ds(base, bpw)], rows_v)
        row_copies = [
            pltpu.async_copy(rows_v.at[pl.ds(j * CHUNK, CHUNK)],
                             mem_hbm.at[idx_v.at[j]], sem_m)
            for j in range(k)
        ]
        for cp in row_copies:
            cp.wait()

    f = pl.kernel(
        body,
        out_type=(),
        mesh=_sc_mesh(),
        scratch_types=[
            pltpu.VMEM((k, CHUNK), jnp.int32),
            pltpu.VMEM((bpw, D), jnp.float32),
            pltpu.SemaphoreType.DMA,
        ],
        compiler_params=pltpu.CompilerParams(needs_layout_passes=False),
        name="sc_scatter_rows",
    )
    f(mem_ref, ids, upd_rows)


def _tc_copy_gru(memory, messages, h, lu_col, ts_col, W_ih, W_hh, b_ih, b_hh):
    M, D = memory.shape
    B, DM = messages.shape
    R = 2048
    G = B // R
    CB = 8192
    NB = M // CB
    bih2 = b_ih.reshape(1, 3 * D)
    bhh2 = b_hh.reshape(1, 3 * D)

    def body(mem_ref, x_ref, h_ref, lu_ref, ts_ref, wih_ref, whh_ref,
             bih_ref, bhh_ref, out_mem_ref, upd_ref):
        i = pl.program_id(0)
        out_mem_ref[...] = mem_ref[...]

        @pl.when(i < G)
        def _():
            x = x_ref[...]
            hh = h_ref[...]
            dn = (((1,), (1,)), ((), ()))
            gi = lax.dot_general(
                x, wih_ref[...], dimension_numbers=dn,
                preferred_element_type=jnp.float32) + bih_ref[...]
            gh = lax.dot_general(
                hh, whh_ref[...], dimension_numbers=dn,
                preferred_element_type=jnp.float32) + bhh_ref[...]
            r = jax.nn.sigmoid(gi[:, :D] + gh[:, :D])
            z = jax.nn.sigmoid(gi[:, D:2 * D] + gh[:, D:2 * D])
            n = jnp.tanh(gi[:, 2 * D:] + r * gh[:, 2 * D:])
            h_new = (1.0 - z) * n + z * hh
            valid = lu_ref[...] <= ts_ref[...]
            upd_ref[...] = jnp.where(valid, h_new, hh)

    return pl.pallas_call(
        body,
        grid=(NB,),
        in_specs=[
            pl.BlockSpec((CB, D), lambda i: (i, 0)),
            pl.BlockSpec((R, DM), lambda i: (jnp.minimum(i, G - 1), 0)),
            pl.BlockSpec((R, D), lambda i: (jnp.minimum(i, G - 1), 0)),
            pl.BlockSpec((R, 1), lambda i: (jnp.minimum(i, G - 1), 0)),
            pl.BlockSpec((R, 1), lambda i: (jnp.minimum(i, G - 1), 0)),
            pl.BlockSpec(W_ih.shape, lambda i: (0, 0)),
            pl.BlockSpec(W_hh.shape, lambda i: (0, 0)),
            pl.BlockSpec((1, 3 * D), lambda i: (0, 0)),
            pl.BlockSpec((1, 3 * D), lambda i: (0, 0)),
        ],
        out_specs=[
            pl.BlockSpec((CB, D), lambda i: (i, 0)),
            pl.BlockSpec((R, D), lambda i: (jnp.minimum(i, G - 1), 0)),
        ],
        out_shape=[
            jax.ShapeDtypeStruct((M, D), jnp.float32),
            jax.ShapeDtypeStruct((B, D), jnp.float32),
        ],
        name="tc_copy_gru",
    )(memory, messages, h, lu_col, ts_col, W_ih, W_hh, bih2, bhh2)


def kernel(memory, last_update, unique_node_ids, unique_messages, timestamps,
           W_ih, W_hh, b_ih, b_hh):
    M = memory.shape[0]
    B = unique_messages.shape[0]
    ids = unique_node_ids.astype(jnp.int32)
    lu_blk = last_update.reshape(M // 128, 128)

    h, lu_g, lu_out_blk = _sc_gather_ts(memory, lu_blk, ids, timestamps)
    lu_col = lu_g.reshape(B, 1)
    ts_col = timestamps.reshape(B, 1)
    out_mem0, upd_rows = _tc_copy_gru(memory, unique_messages, h, lu_col,
                                      ts_col, W_ih, W_hh, b_ih, b_hh)

    mem_ref = jax.new_ref(out_mem0)
    _sc_scatter(mem_ref, ids, upd_rows)
    return mem_ref[...], lu_out_blk.reshape(M)
